# baseline (speedup 1.0000x reference)
import functools
import jax
import jax.numpy as jnp
import numpy as np
from jax import lax
from jax.experimental import pallas as pl
from jax.experimental.pallas import tpu as pltpu
from jax.experimental.pallas import tpu_sc as plsc

N = 10000
E = 160000
D = 256
HD = 256
H = 8
OUT = 256
NBLK = 400

NC = 2
NS = 16
NW = NC * NS
E_PAD = 163840
ET = E_PAD // NW
G_EDGES = 4
CHUNK_G = 128
NCHUNK = ET // (CHUNK_G * G_EDGES)
N_PAD = 10240

RSC = 640
NSWEEP = 8
CB = 5120
NCB = (E_PAD // NS) // CB


def _qkvs_kernel(x_ref, wq_ref, wk_ref, wv_ref, ws_ref,
                 bq_ref, bk_ref, bv_ref, bs_ref,
                 q_ref, k_ref, v_ref, s_ref):
    x = x_ref[...]
    q_ref[...] = (jnp.dot(x, wq_ref[...], preferred_element_type=jnp.float32)
                  + bq_ref[...]).astype(jnp.bfloat16)
    k_ref[...] = (jnp.dot(x, wk_ref[...], preferred_element_type=jnp.float32)
                  + bk_ref[...]).astype(jnp.bfloat16)
    v_ref[...] = jnp.dot(x, wv_ref[...], preferred_element_type=jnp.float32) + bv_ref[...]
    s_ref[...] = jnp.dot(x, ws_ref[...], preferred_element_type=jnp.float32) + bs_ref[...]


def _qkvs_fused_kernel(c_ref, p_ref, wq_ref, wk_ref, wv_ref, ws_ref,
                       bq_ref, bk_ref, bv_ref, bs_ref,
                       q_ref, k_ref, v_ref, s_ref):
    x = jax.nn.relu(c_ref[...] + p_ref[...])
    q_ref[...] = (jnp.dot(x, wq_ref[...], preferred_element_type=jnp.float32)
                  + bq_ref[...]).astype(jnp.bfloat16)
    k_ref[...] = (jnp.dot(x, wk_ref[...], preferred_element_type=jnp.float32)
                  + bk_ref[...]).astype(jnp.bfloat16)
    v_ref[...] = jnp.dot(x, wv_ref[...], preferred_element_type=jnp.float32) + bv_ref[...]
    s_ref[...] = jnp.dot(x, ws_ref[...], preferred_element_type=jnp.float32) + bs_ref[...]


def _qkvs_call(kernel_fn, xs, Wq, bq, Wk, bk, Wv, bv, Ws, bs):
    n = xs[0].shape[0]
    d = Wq.shape[0]
    f = Wq.shape[1]
    fs = Ws.shape[1]
    xspec = [pl.BlockSpec((NBLK, d), lambda i: (i, 0)) for _ in xs]
    wspec = lambda ff: pl.BlockSpec((d, ff), lambda i: (0, 0))
    bspec = lambda ff: pl.BlockSpec((1, ff), lambda i: (0, 0))
    return pl.pallas_call(
        kernel_fn,
        grid=(n // NBLK,),
        in_specs=xspec + [wspec(f), wspec(f), wspec(f), wspec(fs),
                          bspec(f), bspec(f), bspec(f), bspec(fs)],
        out_specs=[pl.BlockSpec((NBLK, f), lambda i: (i, 0)),
                   pl.BlockSpec((NBLK, f), lambda i: (i, 0)),
                   pl.BlockSpec((NBLK, f), lambda i: (i, 0)),
                   pl.BlockSpec((NBLK, fs), lambda i: (i, 0))],
        out_shape=[jax.ShapeDtypeStruct((n, f), jnp.bfloat16),
                   jax.ShapeDtypeStruct((n, f), jnp.bfloat16),
                   jax.ShapeDtypeStruct((n, f), jnp.float32),
                   jax.ShapeDtypeStruct((n, fs), jnp.float32)],
    )(*xs, Wq, Wk, Wv, Ws, bq.reshape(1, f), bk.reshape(1, f),
      bv.reshape(1, f), bs.reshape(1, fs))


def _fused_mm_kernel(c_ref, s_ref, w_ref, b_ref, o_ref):
    h = jax.nn.relu(c_ref[...] + s_ref[...])
    o_ref[...] = jnp.dot(h, w_ref[...],
                         preferred_element_type=jnp.float32) + b_ref[...]


def _fused_matmul(conv, skip, W, b):
    n, d = conv.shape
    f = W.shape[1]
    return pl.pallas_call(
        _fused_mm_kernel,
        grid=(n // NBLK,),
        in_specs=[pl.BlockSpec((NBLK, d), lambda i: (i, 0)),
                  pl.BlockSpec((NBLK, d), lambda i: (i, 0)),
                  pl.BlockSpec((d, f), lambda i: (0, 0)),
                  pl.BlockSpec((1, f), lambda i: (0, 0))],
        out_specs=pl.BlockSpec((NBLK, f), lambda i: (i, 0)),
        out_shape=jax.ShapeDtypeStruct((n, f), jnp.float32),
    )(conv, skip, W, b.reshape(1, f))


_SC_PARAMS = pltpu.CompilerParams(needs_layout_passes=False,
                                  use_tc_tiling_on_sc=False)
MASKHI = np.int32(-65536)


def _phase_a_body(q_hbm, k_hbm, src_hbm, dst_hbm, p_out, den_out,
                  srcb, dstb, qrows, krows, pstage, zbuf, fbuf,
                  den_sh, sem0, sem1):
    cid = lax.axis_index("c")
    sid = lax.axis_index("s")
    wid = sid * NC + cid
    lane = lax.iota(jnp.int32, 16)

    for j in range(128):
        zbuf[j, :] = jnp.zeros((16,), jnp.float32)
    for t in range(5):
        pltpu.sync_copy(zbuf, den_sh.at[pl.ds(sid * 640 + t * 128, 128)])
    plsc.subcore_barrier()

    def chunk_body(c, _):
        gbase = wid * (ET // G_EDGES) + c * CHUNK_G
        pltpu.sync_copy(src_hbm.at[pl.ds(gbase, CHUNK_G)], srcb)
        pltpu.sync_copy(dst_hbm.at[pl.ds(gbase, CHUNK_G)], dstb)

        def issue(g, slot, sem):
            pltpu.async_copy(q_hbm.at[dstb.at[g]], qrows.at[slot], sem)
            pltpu.async_copy(k_hbm.at[srcb.at[g]], krows.at[slot], sem)

        issue(0, 0, sem0)
        issue(1, 1, sem1)

        def pair_body(i, _):
            for slot, sem in ((0, sem0), (1, sem1)):
                g = i * 2 + slot
                pltpu.make_async_copy(q_hbm.at[dstb.at[g]],
                                      qrows.at[slot], sem).wait()
                pltpu.make_async_copy(k_hbm.at[srcb.at[g]],
                                      krows.at[slot], sem).wait()

                for r in range(G_EDGES):
                    pvec = jnp.zeros((16,), jnp.float32)
                    for h in range(H):
                        base = h * HD
                        acc = jnp.zeros((16,), jnp.float32)
                        for j in range(8):
                            sl = pl.ds(base + j * 32, 32)
                            qi = plsc.bitcast(qrows[slot, r, sl], jnp.int32)
                            ki = plsc.bitcast(krows[slot, r, sl], jnp.int32)
                            qh = plsc.bitcast(qi & MASKHI, jnp.float32)
                            kh = plsc.bitcast(ki & MASKHI, jnp.float32)
                            ql = plsc.bitcast(qi << 16, jnp.float32)
                            kl = plsc.bitcast(ki << 16, jnp.float32)
                            acc = acc + qh * kh + ql * kl
                        dot = jnp.sum(acc) * 0.0625
                        pvec = pvec + jnp.where(lane == h, dot, 0.0)
                    e_glob = (wid * ET + c * (CHUNK_G * G_EDGES)
                              + g * G_EDGES + r)
                    keep = (lane < 8) & (e_glob < E)
                    pstage[g, r, :] = jnp.where(keep, jnp.exp(pvec), 0.0)
                pltpu.sync_copy(pstage.at[g], den_sh.at[dstb.at[g]],
                                add=True)

                @pl.when(i * 2 + slot + 2 < CHUNK_G)
                def _():
                    issue(g + 2, slot, sem)
            return 0

        lax.fori_loop(0, CHUNK_G // 2, pair_body, 0)
        pltpu.sync_copy(pstage, p_out.at[pl.ds(gbase, CHUNK_G)])
        return 0

    lax.fori_loop(0, NCHUNK, chunk_body, 0)
    plsc.subcore_barrier()

    for t in range(5):
        sl = pl.ds(sid * 640 + t * 128, 128)
        pltpu.sync_copy(den_sh.at[sl], fbuf)
        pltpu.sync_copy(fbuf, den_out.at[cid].at[sl])


def _phase_a(q, k, src2, dst2):
    mesh = plsc.VectorSubcoreMesh(core_axis_name="c", subcore_axis_name="s")
    f = pl.kernel(
        _phase_a_body,
        out_type=[jax.ShapeDtypeStruct((E_PAD // G_EDGES, G_EDGES, 16),
                                       jnp.float32),
                  jax.ShapeDtypeStruct((NC, N_PAD, 16), jnp.float32)],
        mesh=mesh,
        compiler_params=_SC_PARAMS,
        scratch_types=[
            pltpu.VMEM((CHUNK_G, G_EDGES), jnp.int32),
            pltpu.VMEM((CHUNK_G, G_EDGES), jnp.int32),
            pltpu.VMEM((2, G_EDGES, H * HD), jnp.bfloat16),
            pltpu.VMEM((2, G_EDGES, H * HD), jnp.bfloat16),
            pltpu.VMEM((CHUNK_G, G_EDGES, 16), jnp.float32),
            pltpu.VMEM((128, 16), jnp.float32),
            pltpu.VMEM((128, 16), jnp.float32),
            pltpu.VMEM_SHARED((N_PAD, 16), jnp.float32),
            pltpu.SemaphoreType.DMA,
            pltpu.SemaphoreType.DMA,
        ],
    )
    return f(q, k, src2, dst2)


def _phase_b_body(v_hbm, src_hbm, dst_hbm, p_hbm, den_hbm, conv_out,
                  srcb, dstb, matchb, vrows, cbuf, prows, drows, idxs, sidx,
                  zf, acc_sh, sg0, sg1, ss0, ss1):
    cid = lax.axis_index("c")
    sid = lax.axis_index("s")
    lane = lax.iota(jnp.int32, 16)
    zero16 = jnp.zeros((16,), jnp.float32)

    def sweep_body(swp, _):
        lo = swp * (NC * RSC) + cid * RSC
        hi = lo + RSC

        for j in range(8):
            def zb(t, _, _j=j):
                zf[_j, pl.ds(t * 16, 16)] = zero16
                return 0
            lax.fori_loop(0, HD // 16, zb, 0)

        def zero_blk(t, _):
            pltpu.sync_copy(zf, acc_sh.at[pl.ds((sid + t * NS) * 8, 8)])
            return 0

        lax.fori_loop(0, RSC // 8 // NS, zero_blk, 0)
        plsc.subcore_barrier()

        def chunk_body(c, _):
            ebase = sid * (E_PAD // NS) + c * CB
            pltpu.sync_copy(src_hbm.at[pl.ds(ebase, CB)], srcb)
            pltpu.sync_copy(dst_hbm.at[pl.ds(ebase, CB)], dstb)

            def scan_body(t, cnt):
                dvec = dstb[pl.ds(t * 16, 16)]
                m = (dvec >= lo) & (dvec < hi)
                plsc.store_compressed(matchb.at[pl.ds(cnt, 16)],
                                      lane + t * 16, mask=m)
                return cnt + jnp.sum(jnp.where(m, 1, 0))

            cnt = lax.fori_loop(0, CB // 16, scan_body, 0)
            npair = (cnt + 31) // 32

            def start(g, slot, sg):
                midx_raw = matchb[pl.ds(g * 16, 16)]
                act = (g * 16 + lane) < cnt
                midx = jnp.where(act, midx_raw, 0)
                srcv = plsc.load_gather(srcb, [midx])
                dstv = plsc.load_gather(dstb, [midx])
                idxs[slot, 0, :] = jnp.where(act, srcv, 0)
                idxs[slot, 1, :] = jnp.where(act, midx + ebase, E_PAD - 1)
                idxs[slot, 2, :] = jnp.where(act, dstv - lo, 0)
                idxs[slot, 3, :] = jnp.where(act, dstv, 0)
                idxs[slot, 4, :] = jnp.where(act, dstv + N_PAD, N_PAD)
                pltpu.async_copy(v_hbm.at[idxs.at[slot, 0]],
                                 vrows.at[slot], sg)
                pltpu.async_copy(p_hbm.at[idxs.at[slot, 1]],
                                 prows.at[slot], sg)
                pltpu.async_copy(den_hbm.at[idxs.at[slot, 3]],
                                 drows.at[slot, 0], sg)
                pltpu.async_copy(den_hbm.at[idxs.at[slot, 4]],
                                 drows.at[slot, 1], sg)

            @pl.when(npair > 0)
            def _():
                start(0, 0, sg0)
                start(1, 1, sg1)

            def pair_body(i, _):
                for slot, sg, ss in ((0, sg0, ss0), (1, sg1, ss1)):
                    g = i * 2 + slot
                    pltpu.make_async_copy(v_hbm.at[idxs.at[slot, 0]],
                                          vrows.at[slot], sg).wait()
                    pltpu.make_async_copy(p_hbm.at[idxs.at[slot, 1]],
                                          prows.at[slot], sg).wait()
                    pltpu.make_async_copy(den_hbm.at[idxs.at[slot, 3]],
                                          drows.at[slot, 0], sg).wait()
                    pltpu.make_async_copy(den_hbm.at[idxs.at[slot, 4]],
                                          drows.at[slot, 1], sg).wait()

                    @pl.when(i > 0)
                    def _(_slot=slot, _ss=ss):
                        pltpu.make_async_copy(
                            cbuf.at[_slot],
                            acc_sh.at[sidx.at[_slot]], _ss).wait()

                    def scale_body(r, _, _slot=slot):
                        pv = prows[_slot, r, :]
                        dv = (drows[_slot, 0, r, :] + drows[_slot, 1, r, :]
                              + 1e-16) * 8.0
                        al = pv / dv
                        for j in range(HD // 16):
                            acc = al[0] * vrows[_slot, r, pl.ds(j * 16, 16)]
                            for h in range(1, H):
                                acc = acc + al[h] * vrows[
                                    _slot, r, pl.ds(h * HD + j * 16, 16)]
                            cbuf[_slot, r, pl.ds(j * 16, 16)] = acc
                        return 0

                    lax.fori_loop(0, 16, scale_body, 0)
                    sidx[slot, :] = idxs[slot, 2, :]
                    pltpu.async_copy(cbuf.at[slot],
                                     acc_sh.at[sidx.at[slot]],
                                     ss, add=True)

                    @pl.when(i + 1 < npair)
                    def _(_g=g, _slot=slot, _sg=sg):
                        start(_g + 2, _slot, _sg)
                return 0

            lax.fori_loop(0, npair, pair_body, 0)

            @pl.when(npair > 0)
            def _():
                pltpu.make_async_copy(cbuf.at[0],
                                      acc_sh.at[sidx.at[0]], ss0).wait()
                pltpu.make_async_copy(cbuf.at[1],
                                      acc_sh.at[sidx.at[1]], ss1).wait()
            return 0

        lax.fori_loop(0, NCB, chunk_body, 0)
        plsc.subcore_barrier()

        def flush_blk(t, _):
            blk = sid + t * NS
            row = swp * (NC * RSC) + cid * RSC + blk * 8

            @pl.when(row < N)
            def _():
                pltpu.sync_copy(acc_sh.at[pl.ds(blk * 8, 8)], zf)
                pltpu.sync_copy(zf, conv_out.at[pl.ds(row, 8)])
            return 0

        lax.fori_loop(0, RSC // 8 // NS, flush_blk, 0)
        plsc.subcore_barrier()
        return 0

    lax.fori_loop(0, NSWEEP, sweep_body, 0)


def _phase_b(v, src1, dst1, p1, den2):
    mesh = plsc.VectorSubcoreMesh(core_axis_name="c", subcore_axis_name="s")
    f = pl.kernel(
        _phase_b_body,
        out_type=jax.ShapeDtypeStruct((N, HD), jnp.float32),
        mesh=mesh,
        compiler_params=_SC_PARAMS,
        scratch_types=[
            pltpu.VMEM((CB,), jnp.int32),
            pltpu.VMEM((CB,), jnp.int32),
            pltpu.VMEM((CB + 16,), jnp.int32),
            pltpu.VMEM((2, 16, H * HD), jnp.float32),
            pltpu.VMEM((2, 16, HD), jnp.float32),
            pltpu.VMEM((2, 16, 16), jnp.float32),
            pltpu.VMEM((2, 2, 16, 16), jnp.float32),
            pltpu.VMEM((2, 5, 16), jnp.int32),
            pltpu.VMEM((2, 16), jnp.int32),
            pltpu.VMEM((8, HD), jnp.float32),
            pltpu.VMEM_SHARED((RSC, HD), jnp.float32),
            pltpu.SemaphoreType.DMA,
            pltpu.SemaphoreType.DMA,
            pltpu.SemaphoreType.DMA,
            pltpu.SemaphoreType.DMA,
        ],
    )
    return f(v, src1, dst1, p1, den2)


def _edge_layer(q, k, v, src1, dst1, src2, dst2):
    p, den = _phase_a(q, k, src2, dst2)
    return _phase_b(v, src1, dst1, p.reshape(E_PAD, 16),
                    den.reshape(NC * N_PAD, 16))


def kernel(x, edge_index, Wq0, bq0, Wk0, bk0, Wv0, bv0, Ws0, bs0,
           Wq1, bq1, Wk1, bk1, Wv1, bv1, Ws1, bs1, Wout, bout):
    src = edge_index[0]
    dst = edge_index[1]
    src1 = jnp.pad(src, (0, E_PAD - E))
    dst1 = jnp.pad(dst, (0, E_PAD - E), constant_values=NC * RSC * NSWEEP)
    src2 = src1.reshape(E_PAD // G_EDGES, G_EDGES)
    dst2 = jnp.pad(dst, (0, E_PAD - E)).reshape(E_PAD // G_EDGES, G_EDGES)

    q0, k0, v0, s0 = _qkvs_call(_qkvs_kernel, [x],
                                Wq0, bq0, Wk0, bk0, Wv0, bv0, Ws0, bs0)
    conv0 = _edge_layer(q0, k0, v0, src1, dst1, src2, dst2)

    q1, k1, v1, s1 = _qkvs_call(_qkvs_fused_kernel, [conv0, s0],
                                Wq1, bq1, Wk1, bk1, Wv1, bv1, Ws1, bs1)
    conv1 = _edge_layer(q1, k1, v1, src1, dst1, src2, dst2)

    return _fused_matmul(conv1, s1, Wout, bout)


# speedup vs baseline: 1.0296x; 1.0296x over previous
import functools
import jax
import jax.numpy as jnp
import numpy as np
from jax import lax
from jax.experimental import pallas as pl
from jax.experimental.pallas import tpu as pltpu
from jax.experimental.pallas import tpu_sc as plsc

N = 10000
E = 160000
D = 256
HD = 256
H = 8
OUT = 256
NBLK = 400

NC = 2
NS = 16
NW = NC * NS
E_PAD = 163840
ET = E_PAD // NW
G_EDGES = 4
CHUNK_G = 128
NCHUNK = ET // (CHUNK_G * G_EDGES)
N_PAD = 10240

RSC = 640
NSWEEP = 8
CB = 5120
NCB = (E_PAD // NS) // CB


def _qkvs_kernel(x_ref, wq_ref, wk_ref, wv_ref, ws_ref,
                 bq_ref, bk_ref, bv_ref, bs_ref,
                 q_ref, k_ref, v_ref, s_ref):
    x = x_ref[...]
    q_ref[...] = (jnp.dot(x, wq_ref[...], preferred_element_type=jnp.float32)
                  + bq_ref[...]).astype(jnp.bfloat16)
    k_ref[...] = (jnp.dot(x, wk_ref[...], preferred_element_type=jnp.float32)
                  + bk_ref[...]).astype(jnp.bfloat16)
    v_ref[...] = jnp.dot(x, wv_ref[...], preferred_element_type=jnp.float32) + bv_ref[...]
    s_ref[...] = jnp.dot(x, ws_ref[...], preferred_element_type=jnp.float32) + bs_ref[...]


def _qkvs_fused_kernel(c_ref, p_ref, wq_ref, wk_ref, wv_ref, ws_ref,
                       bq_ref, bk_ref, bv_ref, bs_ref,
                       q_ref, k_ref, v_ref, s_ref):
    x = jax.nn.relu(c_ref[...] + p_ref[...])
    q_ref[...] = (jnp.dot(x, wq_ref[...], preferred_element_type=jnp.float32)
                  + bq_ref[...]).astype(jnp.bfloat16)
    k_ref[...] = (jnp.dot(x, wk_ref[...], preferred_element_type=jnp.float32)
                  + bk_ref[...]).astype(jnp.bfloat16)
    v_ref[...] = jnp.dot(x, wv_ref[...], preferred_element_type=jnp.float32) + bv_ref[...]
    s_ref[...] = jnp.dot(x, ws_ref[...], preferred_element_type=jnp.float32) + bs_ref[...]


def _qkvs_call(kernel_fn, xs, Wq, bq, Wk, bk, Wv, bv, Ws, bs):
    n = xs[0].shape[0]
    d = Wq.shape[0]
    f = Wq.shape[1]
    fs = Ws.shape[1]
    xspec = [pl.BlockSpec((NBLK, d), lambda i: (i, 0)) for _ in xs]
    wspec = lambda ff: pl.BlockSpec((d, ff), lambda i: (0, 0))
    bspec = lambda ff: pl.BlockSpec((1, ff), lambda i: (0, 0))
    return pl.pallas_call(
        kernel_fn,
        grid=(n // NBLK,),
        in_specs=xspec + [wspec(f), wspec(f), wspec(f), wspec(fs),
                          bspec(f), bspec(f), bspec(f), bspec(fs)],
        out_specs=[pl.BlockSpec((NBLK, f), lambda i: (i, 0)),
                   pl.BlockSpec((NBLK, f), lambda i: (i, 0)),
                   pl.BlockSpec((NBLK, f), lambda i: (i, 0)),
                   pl.BlockSpec((NBLK, fs), lambda i: (i, 0))],
        out_shape=[jax.ShapeDtypeStruct((n, f), jnp.bfloat16),
                   jax.ShapeDtypeStruct((n, f), jnp.bfloat16),
                   jax.ShapeDtypeStruct((n, f), jnp.float32),
                   jax.ShapeDtypeStruct((n, fs), jnp.float32)],
    )(*xs, Wq, Wk, Wv, Ws, bq.reshape(1, f), bk.reshape(1, f),
      bv.reshape(1, f), bs.reshape(1, fs))


def _fused_mm_kernel(c_ref, s_ref, w_ref, b_ref, o_ref):
    h = jax.nn.relu(c_ref[...] + s_ref[...])
    o_ref[...] = jnp.dot(h, w_ref[...],
                         preferred_element_type=jnp.float32) + b_ref[...]


def _fused_matmul(conv, skip, W, b):
    n, d = conv.shape
    f = W.shape[1]
    return pl.pallas_call(
        _fused_mm_kernel,
        grid=(n // NBLK,),
        in_specs=[pl.BlockSpec((NBLK, d), lambda i: (i, 0)),
                  pl.BlockSpec((NBLK, d), lambda i: (i, 0)),
                  pl.BlockSpec((d, f), lambda i: (0, 0)),
                  pl.BlockSpec((1, f), lambda i: (0, 0))],
        out_specs=pl.BlockSpec((NBLK, f), lambda i: (i, 0)),
        out_shape=jax.ShapeDtypeStruct((n, f), jnp.float32),
    )(conv, skip, W, b.reshape(1, f))


_SC_PARAMS = pltpu.CompilerParams(needs_layout_passes=False,
                                  use_tc_tiling_on_sc=False)
MASKHI = np.int32(-65536)


def _phase_a_body(q_hbm, k_hbm, src_hbm, dst_hbm, p_out, den_out,
                  srcb, dstb, qrows, krows, pstage, zbuf, fbuf,
                  den_sh, sem0, sem1):
    cid = lax.axis_index("c")
    sid = lax.axis_index("s")
    wid = sid * NC + cid
    lane = lax.iota(jnp.int32, 16)

    for j in range(128):
        zbuf[j, :] = jnp.zeros((16,), jnp.float32)
    for t in range(5):
        pltpu.sync_copy(zbuf, den_sh.at[pl.ds(sid * 640 + t * 128, 128)])
    plsc.subcore_barrier()

    def chunk_body(c, _):
        gbase = wid * (ET // G_EDGES) + c * CHUNK_G
        pltpu.sync_copy(src_hbm.at[pl.ds(gbase, CHUNK_G)], srcb)
        pltpu.sync_copy(dst_hbm.at[pl.ds(gbase, CHUNK_G)], dstb)

        def issue(g, slot, sem):
            pltpu.async_copy(q_hbm.at[dstb.at[g]], qrows.at[slot], sem)
            pltpu.async_copy(k_hbm.at[srcb.at[g]], krows.at[slot], sem)

        issue(0, 0, sem0)
        issue(1, 1, sem1)

        def pair_body(i, _):
            for slot, sem in ((0, sem0), (1, sem1)):
                g = i * 2 + slot
                pltpu.make_async_copy(q_hbm.at[dstb.at[g]],
                                      qrows.at[slot], sem).wait()
                pltpu.make_async_copy(k_hbm.at[srcb.at[g]],
                                      krows.at[slot], sem).wait()

                def edge_body(r, _, _slot=slot, _g=g):
                    pvec = jnp.zeros((16,), jnp.float32)
                    for h in range(H):
                        base = h * HD
                        acc = jnp.zeros((16,), jnp.float32)
                        for j in range(8):
                            sl = pl.ds(base + j * 32, 32)
                            qi = plsc.bitcast(qrows[_slot, r, sl], jnp.int32)
                            ki = plsc.bitcast(krows[_slot, r, sl], jnp.int32)
                            qh = plsc.bitcast(qi & MASKHI, jnp.float32)
                            kh = plsc.bitcast(ki & MASKHI, jnp.float32)
                            ql = plsc.bitcast(qi << 16, jnp.float32)
                            kl = plsc.bitcast(ki << 16, jnp.float32)
                            acc = acc + qh * kh + ql * kl
                        dot = jnp.sum(acc) * 0.0625
                        pvec = pvec + jnp.where(lane == h, dot, 0.0)
                    e_glob = (wid * ET + c * (CHUNK_G * G_EDGES)
                              + _g * G_EDGES + r)
                    keep = (lane < 8) & (e_glob < E)
                    pstage[_g, r, :] = jnp.where(keep, jnp.exp(pvec), 0.0)
                    return 0

                lax.fori_loop(0, G_EDGES, edge_body, 0)
                pltpu.sync_copy(pstage.at[g], den_sh.at[dstb.at[g]],
                                add=True)

                @pl.when(i * 2 + slot + 2 < CHUNK_G)
                def _():
                    issue(g + 2, slot, sem)
            return 0

        lax.fori_loop(0, CHUNK_G // 2, pair_body, 0)
        pltpu.sync_copy(pstage, p_out.at[pl.ds(gbase, CHUNK_G)])
        return 0

    lax.fori_loop(0, NCHUNK, chunk_body, 0)
    plsc.subcore_barrier()

    for t in range(5):
        sl = pl.ds(sid * 640 + t * 128, 128)
        pltpu.sync_copy(den_sh.at[sl], fbuf)
        pltpu.sync_copy(fbuf, den_out.at[cid].at[sl])


def _phase_a(q, k, src2, dst2):
    mesh = plsc.VectorSubcoreMesh(core_axis_name="c", subcore_axis_name="s")
    f = pl.kernel(
        _phase_a_body,
        out_type=[jax.ShapeDtypeStruct((E_PAD // G_EDGES, G_EDGES, 16),
                                       jnp.float32),
                  jax.ShapeDtypeStruct((NC, N_PAD, 16), jnp.float32)],
        mesh=mesh,
        compiler_params=_SC_PARAMS,
        scratch_types=[
            pltpu.VMEM((CHUNK_G, G_EDGES), jnp.int32),
            pltpu.VMEM((CHUNK_G, G_EDGES), jnp.int32),
            pltpu.VMEM((2, G_EDGES, H * HD), jnp.bfloat16),
            pltpu.VMEM((2, G_EDGES, H * HD), jnp.bfloat16),
            pltpu.VMEM((CHUNK_G, G_EDGES, 16), jnp.float32),
            pltpu.VMEM((128, 16), jnp.float32),
            pltpu.VMEM((128, 16), jnp.float32),
            pltpu.VMEM_SHARED((N_PAD, 16), jnp.float32),
            pltpu.SemaphoreType.DMA,
            pltpu.SemaphoreType.DMA,
        ],
    )
    return f(q, k, src2, dst2)


def _phase_b_body(v_hbm, src_hbm, dst_hbm, p_hbm, den_hbm, conv_out,
                  srcb, dstb, matchb, vrows, cbuf, prows, drows, idxs, sidx,
                  zf, acc_sh, sg0, sg1, ss0, ss1):
    cid = lax.axis_index("c")
    sid = lax.axis_index("s")
    lane = lax.iota(jnp.int32, 16)
    zero16 = jnp.zeros((16,), jnp.float32)

    def sweep_body(swp, _):
        lo = swp * (NC * RSC) + cid * RSC
        hi = lo + RSC

        for j in range(8):
            def zb(t, _, _j=j):
                zf[_j, pl.ds(t * 16, 16)] = zero16
                return 0
            lax.fori_loop(0, HD // 16, zb, 0)

        def zero_blk(t, _):
            pltpu.sync_copy(zf, acc_sh.at[pl.ds((sid + t * NS) * 8, 8)])
            return 0

        lax.fori_loop(0, RSC // 8 // NS, zero_blk, 0)
        plsc.subcore_barrier()

        def chunk_body(c, _):
            ebase = sid * (E_PAD // NS) + c * CB
            pltpu.sync_copy(src_hbm.at[pl.ds(ebase, CB)], srcb)
            pltpu.sync_copy(dst_hbm.at[pl.ds(ebase, CB)], dstb)

            def scan_body(t, cnt):
                dvec = dstb[pl.ds(t * 16, 16)]
                m = (dvec >= lo) & (dvec < hi)
                plsc.store_compressed(matchb.at[pl.ds(cnt, 16)],
                                      lane + t * 16, mask=m)
                return cnt + jnp.sum(jnp.where(m, 1, 0))

            cnt = lax.fori_loop(0, CB // 16, scan_body, 0)
            npair = (cnt + 31) // 32

            def start(g, slot, sg):
                midx_raw = matchb[pl.ds(g * 16, 16)]
                act = (g * 16 + lane) < cnt
                midx = jnp.where(act, midx_raw, 0)
                srcv = plsc.load_gather(srcb, [midx])
                dstv = plsc.load_gather(dstb, [midx])
                idxs[slot, 0, :] = jnp.where(act, srcv, 0)
                idxs[slot, 1, :] = jnp.where(act, midx + ebase, E_PAD - 1)
                idxs[slot, 2, :] = jnp.where(act, dstv - lo, 0)
                idxs[slot, 3, :] = jnp.where(act, dstv, 0)
                idxs[slot, 4, :] = jnp.where(act, dstv + N_PAD, N_PAD)
                pltpu.async_copy(v_hbm.at[idxs.at[slot, 0]],
                                 vrows.at[slot], sg)
                pltpu.async_copy(p_hbm.at[idxs.at[slot, 1]],
                                 prows.at[slot], sg)
                pltpu.async_copy(den_hbm.at[idxs.at[slot, 3]],
                                 drows.at[slot, 0], sg)
                pltpu.async_copy(den_hbm.at[idxs.at[slot, 4]],
                                 drows.at[slot, 1], sg)

            @pl.when(npair > 0)
            def _():
                start(0, 0, sg0)
                start(1, 1, sg1)

            def pair_body(i, _):
                for slot, sg, ss in ((0, sg0, ss0), (1, sg1, ss1)):
                    g = i * 2 + slot
                    pltpu.make_async_copy(v_hbm.at[idxs.at[slot, 0]],
                                          vrows.at[slot], sg).wait()
                    pltpu.make_async_copy(p_hbm.at[idxs.at[slot, 1]],
                                          prows.at[slot], sg).wait()
                    pltpu.make_async_copy(den_hbm.at[idxs.at[slot, 3]],
                                          drows.at[slot, 0], sg).wait()
                    pltpu.make_async_copy(den_hbm.at[idxs.at[slot, 4]],
                                          drows.at[slot, 1], sg).wait()

                    @pl.when(i > 0)
                    def _(_slot=slot, _ss=ss):
                        pltpu.make_async_copy(
                            cbuf.at[_slot],
                            acc_sh.at[sidx.at[_slot]], _ss).wait()

                    def scale_body(r, _, _slot=slot):
                        pv = prows[_slot, r, :]
                        dv = (drows[_slot, 0, r, :] + drows[_slot, 1, r, :]
                              + 1e-16) * 8.0
                        al = pv / dv
                        for j in range(HD // 16):
                            acc = al[0] * vrows[_slot, r, pl.ds(j * 16, 16)]
                            for h in range(1, H):
                                acc = acc + al[h] * vrows[
                                    _slot, r, pl.ds(h * HD + j * 16, 16)]
                            cbuf[_slot, r, pl.ds(j * 16, 16)] = acc
                        return 0

                    lax.fori_loop(0, 16, scale_body, 0)
                    sidx[slot, :] = idxs[slot, 2, :]
                    pltpu.async_copy(cbuf.at[slot],
                                     acc_sh.at[sidx.at[slot]],
                                     ss, add=True)

                    @pl.when(i + 1 < npair)
                    def _(_g=g, _slot=slot, _sg=sg):
                        start(_g + 2, _slot, _sg)
                return 0

            lax.fori_loop(0, npair, pair_body, 0)

            @pl.when(npair > 0)
            def _():
                pltpu.make_async_copy(cbuf.at[0],
                                      acc_sh.at[sidx.at[0]], ss0).wait()
                pltpu.make_async_copy(cbuf.at[1],
                                      acc_sh.at[sidx.at[1]], ss1).wait()
            return 0

        lax.fori_loop(0, NCB, chunk_body, 0)
        plsc.subcore_barrier()

        def flush_blk(t, _):
            blk = sid + t * NS
            row = swp * (NC * RSC) + cid * RSC + blk * 8

            @pl.when(row < N)
            def _():
                pltpu.sync_copy(acc_sh.at[pl.ds(blk * 8, 8)], zf)
                pltpu.sync_copy(zf, conv_out.at[pl.ds(row, 8)])
            return 0

        lax.fori_loop(0, RSC // 8 // NS, flush_blk, 0)
        plsc.subcore_barrier()
        return 0

    lax.fori_loop(0, NSWEEP, sweep_body, 0)


def _phase_b(v, src1, dst1, p1, den2):
    mesh = plsc.VectorSubcoreMesh(core_axis_name="c", subcore_axis_name="s")
    f = pl.kernel(
        _phase_b_body,
        out_type=jax.ShapeDtypeStruct((N, HD), jnp.float32),
        mesh=mesh,
        compiler_params=_SC_PARAMS,
        scratch_types=[
            pltpu.VMEM((CB,), jnp.int32),
            pltpu.VMEM((CB,), jnp.int32),
            pltpu.VMEM((CB + 16,), jnp.int32),
            pltpu.VMEM((2, 16, H * HD), jnp.float32),
            pltpu.VMEM((2, 16, HD), jnp.float32),
            pltpu.VMEM((2, 16, 16), jnp.float32),
            pltpu.VMEM((2, 2, 16, 16), jnp.float32),
            pltpu.VMEM((2, 5, 16), jnp.int32),
            pltpu.VMEM((2, 16), jnp.int32),
            pltpu.VMEM((8, HD), jnp.float32),
            pltpu.VMEM_SHARED((RSC, HD), jnp.float32),
            pltpu.SemaphoreType.DMA,
            pltpu.SemaphoreType.DMA,
            pltpu.SemaphoreType.DMA,
            pltpu.SemaphoreType.DMA,
        ],
    )
    return f(v, src1, dst1, p1, den2)


def _edge_layer(q, k, v, src1, dst1, src2, dst2):
    p, den = _phase_a(q, k, src2, dst2)
    return _phase_b(v, src1, dst1, p.reshape(E_PAD, 16),
                    den.reshape(NC * N_PAD, 16))


def kernel(x, edge_index, Wq0, bq0, Wk0, bk0, Wv0, bv0, Ws0, bs0,
           Wq1, bq1, Wk1, bk1, Wv1, bv1, Ws1, bs1, Wout, bout):
    src = edge_index[0]
    dst = edge_index[1]
    src1 = jnp.pad(src, (0, E_PAD - E))
    dst1 = jnp.pad(dst, (0, E_PAD - E), constant_values=NC * RSC * NSWEEP)
    src2 = src1.reshape(E_PAD // G_EDGES, G_EDGES)
    dst2 = jnp.pad(dst, (0, E_PAD - E)).reshape(E_PAD // G_EDGES, G_EDGES)

    q0, k0, v0, s0 = _qkvs_call(_qkvs_kernel, [x],
                                Wq0, bq0, Wk0, bk0, Wv0, bv0, Ws0, bs0)
    conv0 = _edge_layer(q0, k0, v0, src1, dst1, src2, dst2)

    q1, k1, v1, s1 = _qkvs_call(_qkvs_fused_kernel, [conv0, s0],
                                Wq1, bq1, Wk1, bk1, Wv1, bv1, Ws1, bs1)
    conv1 = _edge_layer(q1, k1, v1, src1, dst1, src2, dst2)

    return _fused_matmul(conv1, s1, Wout, bout)
